# baseline (speedup 1.0000x reference)
import functools

import jax
import jax.numpy as jnp
from jax import lax
from jax.experimental import pallas as pl
from jax.experimental.pallas import tpu as pltpu
from jax.experimental.pallas import tpu_sc as plsc

B = 16384
L = 20
D = 64
V = 1000000
_f32 = jnp.float32

_HALF = V // 2
_NC = 2
_NS = 16
_NW = _NC * _NS
_CB = B // _NW
_NB = 32
_KB = _NB * L
_NBLK = _CB // _NB
_CHUNK = 128
_LANES = 16
_PACK_NR = 2048
_LOG_ROWS = B * L // 128


def _pack_kernel(cin_ref, xin_ref, cen_ref, ctx_ref):
    ct = jnp.transpose(cin_ref[...])
    cen_ref[:, 0:D] = ct[0:_PACK_NR]
    cen_ref[:, D:2 * D] = ct[_PACK_NR:2 * _PACK_NR]
    xt = jnp.transpose(xin_ref[...])
    ctx_ref[:, 0:D] = xt[0:_PACK_NR]
    ctx_ref[:, D:2 * D] = xt[_PACK_NR:2 * _PACK_NR]


def _pack_tables(cent_t, ctxt_t):
    nsteps = -(-V // (2 * _PACK_NR))
    in_spec = pl.BlockSpec((D, 2 * _PACK_NR), lambda i: (0, i))
    out_spec = pl.BlockSpec((_PACK_NR, 2 * D), lambda i: (i, 0))
    return pl.pallas_call(
        _pack_kernel,
        grid=(nsteps,),
        in_specs=[in_spec, in_spec],
        out_specs=[out_spec, out_spec],
        out_shape=[
            jax.ShapeDtypeStruct((nsteps * _PACK_NR, 2 * D), _f32),
            jax.ShapeDtypeStruct((nsteps * _PACK_NR, 2 * D), _f32),
        ],
    )(cent_t, ctxt_t)


def _make_sc_logits():
    mesh = plsc.VectorSubcoreMesh(core_axis_name="c", subcore_axis_name="s")

    @functools.partial(
        pl.kernel,
        mesh=mesh,
        compiler_params=pltpu.CompilerParams(
            needs_layout_passes=False, use_tc_tiling_on_sc=True),
        out_type=(
            jax.ShapeDtypeStruct((_LOG_ROWS, 128), _f32),
            jax.ShapeDtypeStruct((_LOG_ROWS, 128), _f32),
        ),
        scratch_types=[
            pltpu.VMEM((_NB,), jnp.int32),
            pltpu.VMEM((_NB,), jnp.int32),
            pltpu.VMEM((_NB + _LANES,), jnp.int32),
            pltpu.VMEM((_NB, 2 * D), _f32),
            pltpu.VMEM((_KB,), jnp.int32),
            pltpu.VMEM((_KB,), jnp.int32),
            pltpu.VMEM((_KB + _LANES,), jnp.int32),
            pltpu.VMEM((_KB, 2 * D), _f32),
            pltpu.VMEM((_LOG_ROWS // _NW, 128), _f32),
            pltpu.VMEM((_LOG_ROWS // _NW, 128), _f32),
            pltpu.SemaphoreType.DMA,
        ],
    )
    def sc_logits(center_hbm, posc_hbm, negc_hbm, cpack_hbm, xpack_hbm,
                  pos_out, neg_out,
                  cidx_v, crow_v, ccol_v, crows_v,
                  kidx_v, krow_v, kcol_v, krows_v, klogp_v, klogn_v, sem):
        wid = lax.axis_index("s") * _NC + lax.axis_index("c")
        base = wid * _CB

        sh = (2 * _PACK_NR).bit_length() - 1
        csh = (_PACK_NR.bit_length() - 1) - 6

        def decode(idx_ref, row_ref, col_ref, n):
            for j in range(n // _LANES):
                sl = pl.ds(j * _LANES, _LANES)
                idx = idx_ref[sl]
                row_ref[sl] = ((idx >> sh) << (sh - 1)) | (idx & (_PACK_NR - 1))
                col_ref[sl] = (idx & _PACK_NR) >> csh

        def blk_body(blk, carry):
            cb_off = base + blk * _NB
            pltpu.sync_copy(center_hbm.at[pl.ds(cb_off, _NB)], cidx_v)
            decode(cidx_v, crow_v, ccol_v, _NB)
            pltpu.async_copy(cpack_hbm.at[crow_v], crows_v, sem).wait()

            for idx_hbm, klog_v in ((posc_hbm, klogp_v), (negc_hbm, klogn_v)):
                off = base * L + blk * _KB
                pltpu.sync_copy(idx_hbm.at[pl.ds(off, _KB)], kidx_v)
                decode(kidx_v, krow_v, kcol_v, _KB)
                gps = [
                    pltpu.async_copy(
                        xpack_hbm.at[krow_v.at[pl.ds(j * _CHUNK, _CHUNK)]],
                        krows_v.at[pl.ds(j * _CHUNK, _CHUNK)], sem)
                    for j in range(_KB // _CHUNK)
                ]
                for gp in gps:
                    gp.wait()

                iota = jnp.arange(_LANES, dtype=jnp.int32)
                for ib0 in range(0, _NB, _LANES):

                    def g_body(j, alogs):
                        i = ib0 + j
                        cc = ccol_v[pl.ds(i, _LANES)][0]
                        cv = [crows_v[i, pl.ds(cc + kk * _LANES, _LANES)]
                              for kk in range(D // _LANES)]
                        lane = iota == j
                        new = []
                        for ll in range(L):
                            r = i * L + ll
                            pc = kcol_v[pl.ds(r, _LANES)][0]
                            acc = krows_v[r, pl.ds(pc, _LANES)] * cv[0]
                            for kk in range(1, D // _LANES):
                                acc = acc + (
                                    krows_v[r, pl.ds(pc + kk * _LANES, _LANES)]
                                    * cv[kk])
                            s = jnp.sum(acc)
                            new.append(jnp.where(
                                lane, jnp.full((_LANES,), s, _f32),
                                alogs[ll]))
                        return tuple(new)

                    alogs = lax.fori_loop(
                        0, _LANES, g_body,
                        tuple(jnp.zeros((_LANES,), _f32) for _ in range(L)))
                    for ll in range(L):
                        pos = ll * _NB + ib0
                        klog_v[blk * (_KB // 128) + pos // 128,
                               pl.ds(pos % 128, _LANES)] = alogs[ll]
            return carry

        lax.fori_loop(0, _NBLK, blk_body, 0)
        tile_rows = _LOG_ROWS // _NW
        pltpu.sync_copy(klogp_v, pos_out.at[pl.ds(wid * tile_rows, tile_rows)])
        pltpu.sync_copy(klogn_v, neg_out.at[pl.ds(wid * tile_rows, tile_rows)])

    return sc_logits


_sc_logits = _make_sc_logits()


def _logsig(x):
    return jnp.where(x > 0, 0.0, x) - jnp.log1p(jnp.exp(-jnp.abs(x)))


def _tc_loss_kernel(p_ref, n_ref, o_ref):
    s = jnp.sum(_logsig(p_ref[...])) + jnp.sum(_logsig(-n_ref[...]))
    o_ref[0, 0] = -s / B


def _tc_loss(pos_log, neg_log):
    return pl.pallas_call(
        _tc_loss_kernel,
        out_shape=jax.ShapeDtypeStruct((1, 1), _f32),
        out_specs=pl.BlockSpec(memory_space=pltpu.SMEM),
    )(pos_log, neg_log)


def kernel(center, pos_c, pos_m, neg_c, neg_m, center_table, context_table):
    del pos_m, neg_m
    cen_packed, ctx_packed = _pack_tables(center_table.T, context_table.T)
    pos_log, neg_log = _sc_logits(
        center, pos_c.reshape(-1), neg_c.reshape(-1), cen_packed, ctx_packed)
    out = _tc_loss(pos_log, neg_log)
    return out[0, 0]


# speedup vs baseline: 1.3755x; 1.3755x over previous
import functools

import jax
import jax.numpy as jnp
from jax import lax
from jax.experimental import pallas as pl
from jax.experimental.pallas import tpu as pltpu
from jax.experimental.pallas import tpu_sc as plsc

B = 16384
L = 20
D = 64
V = 1000000
_f32 = jnp.float32

_NC = 2
_NS = 16
_NW = _NC * _NS
_CB = B // _NW
_NB = 32
_KB = _NB * L
_NBLK = _CB // _NB
_CHUNK = 128
_LANES = 16
_PACK_W = 4096
_LOG_ROWS = B * L // 128


def _pack_kernel(cin_ref, xin_ref, out_ref):
    both = jnp.concatenate([cin_ref[...], xin_ref[...]], axis=0)
    out_ref[...] = jnp.transpose(both)


def _pack_tables(cent_t, ctxt_t):
    nsteps = -(-V // _PACK_W)
    in_spec = pl.BlockSpec((D, _PACK_W), lambda i: (0, i))
    out_spec = pl.BlockSpec((_PACK_W, 2 * D), lambda i: (i, 0))
    return pl.pallas_call(
        _pack_kernel,
        grid=(nsteps,),
        in_specs=[in_spec, in_spec],
        out_specs=out_spec,
        out_shape=jax.ShapeDtypeStruct((nsteps * _PACK_W, 2 * D), _f32),
    )(cent_t, ctxt_t)


def _make_sc_logits():
    mesh = plsc.VectorSubcoreMesh(core_axis_name="c", subcore_axis_name="s")

    @functools.partial(
        pl.kernel,
        mesh=mesh,
        compiler_params=pltpu.CompilerParams(
            needs_layout_passes=False, use_tc_tiling_on_sc=True),
        out_type=(
            jax.ShapeDtypeStruct((_LOG_ROWS, 128), _f32),
            jax.ShapeDtypeStruct((_LOG_ROWS, 128), _f32),
        ),
        scratch_types=[
            pltpu.VMEM((_NB,), jnp.int32),
            pltpu.VMEM((_NB, 2 * D), _f32),
            pltpu.VMEM((_KB,), jnp.int32),
            pltpu.VMEM((_KB, 2 * D), _f32),
            pltpu.VMEM((_LOG_ROWS // _NW, 128), _f32),
            pltpu.VMEM((_LOG_ROWS // _NW, 128), _f32),
            pltpu.SemaphoreType.DMA,
        ],
    )
    def sc_logits(center_hbm, posc_hbm, negc_hbm, pack_hbm,
                  pos_out, neg_out,
                  cidx_v, crows_v, kidx_v, krows_v, klogp_v, klogn_v, sem):
        wid = lax.axis_index("s") * _NC + lax.axis_index("c")
        base = wid * _CB

        def blk_body(blk, carry):
            cb_off = base + blk * _NB
            pltpu.sync_copy(center_hbm.at[pl.ds(cb_off, _NB)], cidx_v)
            pltpu.async_copy(pack_hbm.at[cidx_v], crows_v, sem).wait()

            for idx_hbm, klog_v in ((posc_hbm, klogp_v), (negc_hbm, klogn_v)):
                off = base * L + blk * _KB
                pltpu.sync_copy(idx_hbm.at[pl.ds(off, _KB)], kidx_v)
                gps = [
                    pltpu.async_copy(
                        pack_hbm.at[kidx_v.at[pl.ds(j * _CHUNK, _CHUNK)]],
                        krows_v.at[pl.ds(j * _CHUNK, _CHUNK)], sem)
                    for j in range(_KB // _CHUNK)
                ]
                for gp in gps:
                    gp.wait()

                iota = jnp.arange(_LANES, dtype=jnp.int32)
                for ib0 in range(0, _NB, _LANES):

                    def g_body(j, alogs):
                        i = ib0 + j
                        cv = [crows_v[i, pl.ds(kk * _LANES, _LANES)]
                              for kk in range(D // _LANES)]
                        lane = iota == j
                        new = []
                        for ll in range(L):
                            r = i * L + ll
                            acc = krows_v[r, pl.ds(D, _LANES)] * cv[0]
                            for kk in range(1, D // _LANES):
                                acc = acc + (
                                    krows_v[r, pl.ds(D + kk * _LANES, _LANES)]
                                    * cv[kk])
                            s = jnp.sum(acc)
                            new.append(jnp.where(
                                lane, jnp.full((_LANES,), s, _f32),
                                alogs[ll]))
                        return tuple(new)

                    alogs = lax.fori_loop(
                        0, _LANES, g_body,
                        tuple(jnp.zeros((_LANES,), _f32) for _ in range(L)))
                    for ll in range(L):
                        pos = ll * _NB + ib0
                        klog_v[blk * (_KB // 128) + pos // 128,
                               pl.ds(pos % 128, _LANES)] = alogs[ll]
            return carry

        lax.fori_loop(0, _NBLK, blk_body, 0)
        tile_rows = _LOG_ROWS // _NW
        pltpu.sync_copy(klogp_v, pos_out.at[pl.ds(wid * tile_rows, tile_rows)])
        pltpu.sync_copy(klogn_v, neg_out.at[pl.ds(wid * tile_rows, tile_rows)])

    return sc_logits


_sc_logits = _make_sc_logits()


def _logsig(x):
    return jnp.where(x > 0, 0.0, x) - jnp.log1p(jnp.exp(-jnp.abs(x)))


def _tc_loss_kernel(p_ref, n_ref, o_ref):
    s = jnp.sum(_logsig(p_ref[...])) + jnp.sum(_logsig(-n_ref[...]))
    o_ref[0, 0] = -s / B


def _tc_loss(pos_log, neg_log):
    return pl.pallas_call(
        _tc_loss_kernel,
        out_shape=jax.ShapeDtypeStruct((1, 1), _f32),
        out_specs=pl.BlockSpec(memory_space=pltpu.SMEM),
    )(pos_log, neg_log)


def kernel(center, pos_c, pos_m, neg_c, neg_m, center_table, context_table):
    del pos_m, neg_m
    packed = _pack_tables(center_table.T, context_table.T)
    pos_log, neg_log = _sc_logits(
        center, pos_c.reshape(-1), neg_c.reshape(-1), packed)
    out = _tc_loss(pos_log, neg_log)
    return out[0, 0]


# speedup vs baseline: 1.5424x; 1.1213x over previous
import functools

import jax
import jax.numpy as jnp
from jax import lax
from jax.experimental import pallas as pl
from jax.experimental.pallas import tpu as pltpu
from jax.experimental.pallas import tpu_sc as plsc

B = 16384
L = 20
D = 64
V = 1000000
_f32 = jnp.float32

_NC = 2
_NS = 16
_NW = _NC * _NS
_CB = B // _NW
_NB = 16
_KB = _NB * L
_NBLK = _CB // _NB
_CHUNKS = ((0, 128), (128, 128), (256, 64))
_LANES = 16
_PACK_W = 4096
_LOG_ROWS = B * L // 128


def _pack_kernel(cin_ref, xin_ref, out_ref):
    both = jnp.concatenate([cin_ref[...], xin_ref[...]], axis=0)
    out_ref[...] = jnp.transpose(both)


def _pack_tables(cent_t, ctxt_t):
    nsteps = -(-V // _PACK_W)
    in_spec = pl.BlockSpec((D, _PACK_W), lambda i: (0, i))
    out_spec = pl.BlockSpec((_PACK_W, 2 * D), lambda i: (i, 0))
    return pl.pallas_call(
        _pack_kernel,
        grid=(nsteps,),
        in_specs=[in_spec, in_spec],
        out_specs=out_spec,
        out_shape=jax.ShapeDtypeStruct((nsteps * _PACK_W, 2 * D), _f32),
    )(cent_t, ctxt_t)


def _make_sc_logits():
    mesh = plsc.VectorSubcoreMesh(core_axis_name="c", subcore_axis_name="s")

    @functools.partial(
        pl.kernel,
        mesh=mesh,
        compiler_params=pltpu.CompilerParams(
            needs_layout_passes=False, use_tc_tiling_on_sc=True),
        out_type=(
            jax.ShapeDtypeStruct((_LOG_ROWS, 128), _f32),
            jax.ShapeDtypeStruct((_LOG_ROWS, 128), _f32),
        ),
        scratch_types=[
            pltpu.VMEM((_NB,), jnp.int32),
            pltpu.VMEM((_NB, 2 * D), _f32),
            pltpu.VMEM((_NB, 2 * D), _f32),
            pltpu.VMEM((_KB,), jnp.int32),
            pltpu.VMEM((_KB,), jnp.int32),
            pltpu.VMEM((_KB, 2 * D), _f32),
            pltpu.VMEM((_KB, 2 * D), _f32),
            pltpu.VMEM((_LOG_ROWS // _NW, 128), _f32),
            pltpu.VMEM((_LOG_ROWS // _NW, 128), _f32),
            pltpu.SemaphoreType.DMA,
            pltpu.SemaphoreType.DMA,
            pltpu.SemaphoreType.DMA,
        ],
    )
    def sc_logits(center_hbm, posc_hbm, negc_hbm, pack_hbm,
                  pos_out, neg_out,
                  cidx_v, crowsa_v, crowsb_v, kidxp_v, kidxn_v,
                  krowsp_v, krowsn_v, klogp_v, klogn_v,
                  semp, semn, semc):
        wid = lax.axis_index("s") * _NC + lax.axis_index("c")
        base = wid * _CB

        def issue_ctx(idx_hbm, blk, kidx_v, krows_v, sem):
            off = base * L + blk * _KB
            pltpu.sync_copy(idx_hbm.at[pl.ds(off, _KB)], kidx_v)
            for o, n in _CHUNKS:
                pltpu.async_copy(
                    pack_hbm.at[kidx_v.at[pl.ds(o, n)]],
                    krows_v.at[pl.ds(o, n)], sem)

        def drain_ctx(krows_v, sem):
            for o, n in _CHUNKS:
                pltpu.make_async_copy(
                    pack_hbm.at[pl.ds(0, n)],
                    krows_v.at[pl.ds(o, n)], sem).wait()

        def issue_cen(blk, crows_v):
            pltpu.sync_copy(
                center_hbm.at[pl.ds(base + blk * _NB, _NB)], cidx_v)
            pltpu.async_copy(pack_hbm.at[cidx_v], crows_v, semc)

        def drain_cen(crows_v):
            pltpu.make_async_copy(
                pack_hbm.at[pl.ds(0, _NB)], crows_v, semc).wait()

        def compute(blk, krows_v, crows_v, klog_v):
            iota = jnp.arange(_LANES, dtype=jnp.int32)

            def g_body(i, alogs):
                cv = [crows_v[i, pl.ds(kk * _LANES, _LANES)]
                      for kk in range(D // _LANES)]
                lane = iota == i
                new = []
                for ll in range(L):
                    r = i * L + ll
                    acc = krows_v[r, pl.ds(D, _LANES)] * cv[0]
                    for kk in range(1, D // _LANES):
                        acc = acc + (
                            krows_v[r, pl.ds(D + kk * _LANES, _LANES)]
                            * cv[kk])
                    s = jnp.sum(acc)
                    new.append(jnp.where(
                        lane, jnp.full((_LANES,), s, _f32), alogs[ll]))
                return tuple(new)

            alogs = lax.fori_loop(
                0, _LANES, g_body,
                tuple(jnp.zeros((_LANES,), _f32) for _ in range(L)))
            for ll in range(L):
                pos = blk * _KB + ll * _NB
                klog_v[pos >> 7, pl.ds(pos & 127, _LANES)] = alogs[ll]

        issue_cen(0, crowsa_v)
        issue_ctx(posc_hbm, 0, kidxp_v, krowsp_v, semp)

        def s_body(s, carry):
            b0 = 2 * s
            b1 = 2 * s + 1
            nb0 = (2 * s + 2) & (_NBLK - 1)

            issue_ctx(negc_hbm, b0, kidxn_v, krowsn_v, semn)
            drain_cen(crowsa_v)
            drain_ctx(krowsp_v, semp)
            compute(b0, krowsp_v, crowsa_v, klogp_v)
            issue_cen(b1, crowsb_v)
            issue_ctx(posc_hbm, b1, kidxp_v, krowsp_v, semp)
            drain_ctx(krowsn_v, semn)
            compute(b0, krowsn_v, crowsa_v, klogn_v)

            issue_ctx(negc_hbm, b1, kidxn_v, krowsn_v, semn)
            drain_cen(crowsb_v)
            drain_ctx(krowsp_v, semp)
            compute(b1, krowsp_v, crowsb_v, klogp_v)
            issue_cen(nb0, crowsa_v)
            issue_ctx(posc_hbm, nb0, kidxp_v, krowsp_v, semp)
            drain_ctx(krowsn_v, semn)
            compute(b1, krowsn_v, crowsb_v, klogn_v)
            return carry

        lax.fori_loop(0, _NBLK // 2, s_body, 0)
        drain_cen(crowsa_v)
        drain_ctx(krowsp_v, semp)

        tile_rows = _LOG_ROWS // _NW
        pltpu.sync_copy(klogp_v, pos_out.at[pl.ds(wid * tile_rows, tile_rows)])
        pltpu.sync_copy(klogn_v, neg_out.at[pl.ds(wid * tile_rows, tile_rows)])

    return sc_logits


_sc_logits = _make_sc_logits()


def _logsig(x):
    return jnp.where(x > 0, 0.0, x) - jnp.log1p(jnp.exp(-jnp.abs(x)))


def _tc_loss_kernel(p_ref, n_ref, o_ref):
    s = jnp.sum(_logsig(p_ref[...])) + jnp.sum(_logsig(-n_ref[...]))
    o_ref[0, 0] = -s / B


def _tc_loss(pos_log, neg_log):
    return pl.pallas_call(
        _tc_loss_kernel,
        out_shape=jax.ShapeDtypeStruct((1, 1), _f32),
        out_specs=pl.BlockSpec(memory_space=pltpu.SMEM),
    )(pos_log, neg_log)


def kernel(center, pos_c, pos_m, neg_c, neg_m, center_table, context_table):
    del pos_m, neg_m
    packed = _pack_tables(center_table.T, context_table.T)
    pos_log, neg_log = _sc_logits(
        center, pos_c.reshape(-1), neg_c.reshape(-1), packed)
    out = _tc_loss(pos_log, neg_log)
    return out[0, 0]


# speedup vs baseline: 1.7051x; 1.1055x over previous
import functools

import jax
import jax.numpy as jnp
from jax import lax
from jax.experimental import pallas as pl
from jax.experimental.pallas import tpu as pltpu
from jax.experimental.pallas import tpu_sc as plsc

B = 16384
L = 20
D = 64
V = 1000000
_f32 = jnp.float32

_NC = 2
_NS = 16
_NW = _NC * _NS
_CB = B // _NW
_NB = 16
_KB = _NB * L
_NBLK = _CB // _NB
_CHUNKS = ((0, 128), (128, 128), (256, 64))
_LANES = 16
_PACK_W = 8192
_LOG_ROWS = B * L // 128


def _pack_kernel(cin_ref, xin_ref, out_ref):
    both = jnp.concatenate([cin_ref[...], xin_ref[...]], axis=0)
    out_ref[...] = jnp.transpose(both)


def _pack_tables(cent_t, ctxt_t):
    nsteps = -(-V // _PACK_W)
    in_spec = pl.BlockSpec((D, _PACK_W), lambda i: (0, i))
    out_spec = pl.BlockSpec((_PACK_W, 2 * D), lambda i: (i, 0))
    return pl.pallas_call(
        _pack_kernel,
        grid=(nsteps,),
        in_specs=[in_spec, in_spec],
        out_specs=out_spec,
        out_shape=jax.ShapeDtypeStruct((nsteps * _PACK_W, 2 * D), _f32),
    )(cent_t, ctxt_t)


def _make_sc_logits():
    mesh = plsc.VectorSubcoreMesh(core_axis_name="c", subcore_axis_name="s")

    @functools.partial(
        pl.kernel,
        mesh=mesh,
        compiler_params=pltpu.CompilerParams(
            needs_layout_passes=False, use_tc_tiling_on_sc=True),
        out_type=(
            jax.ShapeDtypeStruct((_LOG_ROWS, 128), _f32),
            jax.ShapeDtypeStruct((_LOG_ROWS, 128), _f32),
        ),
        scratch_types=[
            pltpu.VMEM((_NB,), jnp.int32),
            pltpu.VMEM((_NB, 2 * D), _f32),
            pltpu.VMEM((_NB, 2 * D), _f32),
            pltpu.VMEM((_KB,), jnp.int32),
            pltpu.VMEM((_KB,), jnp.int32),
            pltpu.VMEM((_KB, 2 * D), _f32),
            pltpu.VMEM((_KB, 2 * D), _f32),
            pltpu.VMEM((_LOG_ROWS // _NW, 128), _f32),
            pltpu.VMEM((_LOG_ROWS // _NW, 128), _f32),
            pltpu.SemaphoreType.DMA,
            pltpu.SemaphoreType.DMA,
            pltpu.SemaphoreType.DMA,
        ],
    )
    def sc_logits(center_hbm, posc_hbm, negc_hbm, pack_hbm,
                  pos_out, neg_out,
                  cidx_v, crowsa_v, crowsb_v, kidxp_v, kidxn_v,
                  krowsp_v, krowsn_v, klogp_v, klogn_v,
                  semp, semn, semc):
        wid = lax.axis_index("s") * _NC + lax.axis_index("c")
        base = wid * _CB

        def issue_ctx(idx_hbm, blk, kidx_v, krows_v, sem):
            off = base * L + blk * _KB
            pltpu.sync_copy(idx_hbm.at[pl.ds(off, _KB)], kidx_v)
            for o, n in _CHUNKS:
                pltpu.async_copy(
                    pack_hbm.at[kidx_v.at[pl.ds(o, n)]],
                    krows_v.at[pl.ds(o, n)], sem)

        def drain_ctx(krows_v, sem):
            for o, n in _CHUNKS:
                pltpu.make_async_copy(
                    pack_hbm.at[pl.ds(0, n)],
                    krows_v.at[pl.ds(o, n)], sem).wait()

        def issue_cen(blk, crows_v):
            pltpu.sync_copy(
                center_hbm.at[pl.ds(base + blk * _NB, _NB)], cidx_v)
            pltpu.async_copy(pack_hbm.at[cidx_v], crows_v, semc)

        def drain_cen(crows_v):
            pltpu.make_async_copy(
                pack_hbm.at[pl.ds(0, _NB)], crows_v, semc).wait()

        def compute(blk, krows_v, crows_v, klog_v):
            iota = jnp.arange(_LANES, dtype=jnp.int32)

            def g_body(i, alogs):
                cv = [crows_v[i, pl.ds(kk * _LANES, _LANES)]
                      for kk in range(D // _LANES)]
                lane = iota == i
                new = []
                for ll in range(L):
                    r = i * L + ll
                    acc = krows_v[r, pl.ds(D, _LANES)] * cv[0]
                    for kk in range(1, D // _LANES):
                        acc = acc + (
                            krows_v[r, pl.ds(D + kk * _LANES, _LANES)]
                            * cv[kk])
                    s = jnp.sum(acc)
                    new.append(jnp.where(
                        lane, jnp.full((_LANES,), s, _f32), alogs[ll]))
                return tuple(new)

            alogs = lax.fori_loop(
                0, _LANES, g_body,
                tuple(jnp.zeros((_LANES,), _f32) for _ in range(L)))
            for ll in range(L):
                pos = blk * _KB + ll * _NB
                klog_v[pos >> 7, pl.ds(pos & 127, _LANES)] = alogs[ll]

        issue_cen(0, crowsa_v)
        issue_ctx(posc_hbm, 0, kidxp_v, krowsp_v, semp)

        def s_body(s, carry):
            b0 = 2 * s
            b1 = 2 * s + 1
            nb0 = (2 * s + 2) & (_NBLK - 1)

            issue_ctx(negc_hbm, b0, kidxn_v, krowsn_v, semn)
            drain_cen(crowsa_v)
            drain_ctx(krowsp_v, semp)
            compute(b0, krowsp_v, crowsa_v, klogp_v)
            issue_cen(b1, crowsb_v)
            issue_ctx(posc_hbm, b1, kidxp_v, krowsp_v, semp)
            drain_ctx(krowsn_v, semn)
            compute(b0, krowsn_v, crowsa_v, klogn_v)

            issue_ctx(negc_hbm, b1, kidxn_v, krowsn_v, semn)
            drain_cen(crowsb_v)
            drain_ctx(krowsp_v, semp)
            compute(b1, krowsp_v, crowsb_v, klogp_v)
            issue_cen(nb0, crowsa_v)
            issue_ctx(posc_hbm, nb0, kidxp_v, krowsp_v, semp)
            drain_ctx(krowsn_v, semn)
            compute(b1, krowsn_v, crowsb_v, klogn_v)
            return carry

        lax.fori_loop(0, _NBLK // 2, s_body, 0)
        drain_cen(crowsa_v)
        drain_ctx(krowsp_v, semp)

        tile_rows = _LOG_ROWS // _NW
        pltpu.sync_copy(klogp_v, pos_out.at[pl.ds(wid * tile_rows, tile_rows)])
        pltpu.sync_copy(klogn_v, neg_out.at[pl.ds(wid * tile_rows, tile_rows)])

    return sc_logits


_sc_logits = _make_sc_logits()


def _logsig(x):
    return jnp.where(x > 0, 0.0, x) - jnp.log1p(jnp.exp(-jnp.abs(x)))


def _tc_loss_kernel(p_ref, n_ref, o_ref):
    s = jnp.sum(_logsig(p_ref[...])) + jnp.sum(_logsig(-n_ref[...]))
    o_ref[0, 0] = -s / B


def _tc_loss(pos_log, neg_log):
    return pl.pallas_call(
        _tc_loss_kernel,
        out_shape=jax.ShapeDtypeStruct((1, 1), _f32),
        out_specs=pl.BlockSpec(memory_space=pltpu.SMEM),
    )(pos_log, neg_log)


def kernel(center, pos_c, pos_m, neg_c, neg_m, center_table, context_table):
    del pos_m, neg_m
    packed = _pack_tables(center_table.T, context_table.T)
    pos_log, neg_log = _sc_logits(
        center, pos_c.reshape(-1), neg_c.reshape(-1), packed)
    out = _tc_loss(pos_log, neg_log)
    return out[0, 0]


# speedup vs baseline: 1.7261x; 1.0123x over previous
import functools

import jax
import jax.numpy as jnp
from jax import lax
from jax.experimental import pallas as pl
from jax.experimental.pallas import tpu as pltpu
from jax.experimental.pallas import tpu_sc as plsc

B = 16384
L = 20
D = 64
V = 1000000
_f32 = jnp.float32

_NC = 2
_NS = 16
_NW = _NC * _NS
_CB = B // _NW
_NB = 16
_KB = _NB * L
_NBLK = _CB // _NB
_CHUNKS = ((0, 128), (128, 128), (256, 64))
_LANES = 16
_PACK_W = 16384
_LOG_ROWS = B * L // 128


def _pack_kernel(cin_ref, xin_ref, out_ref):
    both = jnp.concatenate([cin_ref[...], xin_ref[...]], axis=0)
    out_ref[...] = jnp.transpose(both)


def _pack_tables(cent_t, ctxt_t):
    nsteps = -(-V // _PACK_W)
    in_spec = pl.BlockSpec((D, _PACK_W), lambda i: (0, i))
    out_spec = pl.BlockSpec((_PACK_W, 2 * D), lambda i: (i, 0))
    return pl.pallas_call(
        _pack_kernel,
        grid=(nsteps,),
        in_specs=[in_spec, in_spec],
        out_specs=out_spec,
        out_shape=jax.ShapeDtypeStruct((nsteps * _PACK_W, 2 * D), _f32),
    )(cent_t, ctxt_t)


def _make_sc_logits():
    mesh = plsc.VectorSubcoreMesh(core_axis_name="c", subcore_axis_name="s")

    @functools.partial(
        pl.kernel,
        mesh=mesh,
        compiler_params=pltpu.CompilerParams(
            needs_layout_passes=False, use_tc_tiling_on_sc=True),
        out_type=(
            jax.ShapeDtypeStruct((_LOG_ROWS, 128), _f32),
            jax.ShapeDtypeStruct((_LOG_ROWS, 128), _f32),
        ),
        scratch_types=[
            pltpu.VMEM((_NB,), jnp.int32),
            pltpu.VMEM((_NB, 2 * D), _f32),
            pltpu.VMEM((_NB, 2 * D), _f32),
            pltpu.VMEM((_KB,), jnp.int32),
            pltpu.VMEM((_KB,), jnp.int32),
            pltpu.VMEM((_KB, 2 * D), _f32),
            pltpu.VMEM((_KB, 2 * D), _f32),
            pltpu.VMEM((_LOG_ROWS // _NW, 128), _f32),
            pltpu.VMEM((_LOG_ROWS // _NW, 128), _f32),
            pltpu.SemaphoreType.DMA,
            pltpu.SemaphoreType.DMA,
            pltpu.SemaphoreType.DMA,
        ],
    )
    def sc_logits(center_hbm, posc_hbm, negc_hbm, pack_hbm,
                  pos_out, neg_out,
                  cidx_v, crowsa_v, crowsb_v, kidxp_v, kidxn_v,
                  krowsp_v, krowsn_v, klogp_v, klogn_v,
                  semp, semn, semc):
        wid = lax.axis_index("s") * _NC + lax.axis_index("c")
        base = wid * _CB

        def issue_ctx(idx_hbm, blk, kidx_v, krows_v, sem):
            off = base * L + blk * _KB
            pltpu.sync_copy(idx_hbm.at[pl.ds(off, _KB)], kidx_v)
            for o, n in _CHUNKS:
                pltpu.async_copy(
                    pack_hbm.at[kidx_v.at[pl.ds(o, n)]],
                    krows_v.at[pl.ds(o, n)], sem)

        def drain_ctx(krows_v, sem):
            for o, n in _CHUNKS:
                pltpu.make_async_copy(
                    pack_hbm.at[pl.ds(0, n)],
                    krows_v.at[pl.ds(o, n)], sem).wait()

        def issue_cen(blk, crows_v):
            pltpu.sync_copy(
                center_hbm.at[pl.ds(base + blk * _NB, _NB)], cidx_v)
            pltpu.async_copy(pack_hbm.at[cidx_v], crows_v, semc)

        def drain_cen(crows_v):
            pltpu.make_async_copy(
                pack_hbm.at[pl.ds(0, _NB)], crows_v, semc).wait()

        def compute(blk, krows_v, crows_v, klog_v):
            iota = jnp.arange(_LANES, dtype=jnp.int32)

            def g_body(i, alogs):
                cv = [crows_v[i, pl.ds(kk * _LANES, _LANES)]
                      for kk in range(D // _LANES)]
                lane = iota == i
                new = []
                for ll in range(L):
                    r = i * L + ll
                    acc = krows_v[r, pl.ds(D, _LANES)] * cv[0]
                    for kk in range(1, D // _LANES):
                        acc = acc + (
                            krows_v[r, pl.ds(D + kk * _LANES, _LANES)]
                            * cv[kk])
                    s = jnp.sum(acc)
                    new.append(jnp.where(
                        lane, jnp.full((_LANES,), s, _f32), alogs[ll]))
                return tuple(new)

            alogs = lax.fori_loop(
                0, _LANES, g_body,
                tuple(jnp.zeros((_LANES,), _f32) for _ in range(L)))
            for ll in range(L):
                pos = blk * _KB + ll * _NB
                klog_v[pos >> 7, pl.ds(pos & 127, _LANES)] = alogs[ll]

        issue_cen(0, crowsa_v)
        issue_ctx(posc_hbm, 0, kidxp_v, krowsp_v, semp)

        def s_body(s, carry):
            b0 = 2 * s
            b1 = 2 * s + 1
            nb0 = (2 * s + 2) & (_NBLK - 1)

            issue_ctx(negc_hbm, b0, kidxn_v, krowsn_v, semn)
            drain_cen(crowsa_v)
            drain_ctx(krowsp_v, semp)
            compute(b0, krowsp_v, crowsa_v, klogp_v)
            issue_cen(b1, crowsb_v)
            issue_ctx(posc_hbm, b1, kidxp_v, krowsp_v, semp)
            drain_ctx(krowsn_v, semn)
            compute(b0, krowsn_v, crowsa_v, klogn_v)

            issue_ctx(negc_hbm, b1, kidxn_v, krowsn_v, semn)
            drain_cen(crowsb_v)
            drain_ctx(krowsp_v, semp)
            compute(b1, krowsp_v, crowsb_v, klogp_v)
            issue_cen(nb0, crowsa_v)
            issue_ctx(posc_hbm, nb0, kidxp_v, krowsp_v, semp)
            drain_ctx(krowsn_v, semn)
            compute(b1, krowsn_v, crowsb_v, klogn_v)
            return carry

        lax.fori_loop(0, _NBLK // 2, s_body, 0)
        drain_cen(crowsa_v)
        drain_ctx(krowsp_v, semp)

        tile_rows = _LOG_ROWS // _NW
        pltpu.sync_copy(klogp_v, pos_out.at[pl.ds(wid * tile_rows, tile_rows)])
        pltpu.sync_copy(klogn_v, neg_out.at[pl.ds(wid * tile_rows, tile_rows)])

    return sc_logits


_sc_logits = _make_sc_logits()


def _logsig(x):
    return jnp.where(x > 0, 0.0, x) - jnp.log1p(jnp.exp(-jnp.abs(x)))


def _tc_loss_kernel(p_ref, n_ref, o_ref):
    s = jnp.sum(_logsig(p_ref[...])) + jnp.sum(_logsig(-n_ref[...]))
    o_ref[0, 0] = -s / B


def _tc_loss(pos_log, neg_log):
    return pl.pallas_call(
        _tc_loss_kernel,
        out_shape=jax.ShapeDtypeStruct((1, 1), _f32),
        out_specs=pl.BlockSpec(memory_space=pltpu.SMEM),
    )(pos_log, neg_log)


def kernel(center, pos_c, pos_m, neg_c, neg_m, center_table, context_table):
    del pos_m, neg_m
    packed = _pack_tables(center_table.T, context_table.T)
    pos_log, neg_log = _sc_logits(
        center, pos_c.reshape(-1), neg_c.reshape(-1), packed)
    out = _tc_loss(pos_log, neg_log)
    return out[0, 0]
